# baseline (speedup 1.0000x reference)
import jax
import jax.numpy as jnp
from jax import lax
from jax.experimental import pallas as pl
from jax.experimental.pallas import tpu as pltpu
from jax.experimental.pallas import tpu_sc as plsc

N = 10000
E = 160000
D = 256
HALF = 128

NT = 16
CH = 80
EPT = E // NT
NCH = EPT // CH
SEG = 5
SEGC = NCH // SEG
RCH = 80
ZNC = N // RCH
RROUNDS = -(-ZNC // NT)

_MESH = plsc.VectorSubcoreMesh(core_axis_name="c", subcore_axis_name="s")


def _deg_body(src_hbm, dst_hbm, odeg_hbm, ideg_hbm, idx_v, ones_v, stage_v,
              deg_s):
    cid = lax.axis_index("c")
    tid = lax.axis_index("s")

    for j in range(CH // 16):
        ones_v[pl.ds(j * 16, 16)] = jnp.ones((16,), jnp.float32)

    @pl.when(tid == 0)
    def _():
        def _zero_stage(i, carry):
            stage_v[pl.ds(i * 16, 16)] = jnp.zeros((16,), jnp.float32)
            return carry

        lax.fori_loop(0, N // 16, _zero_stage, 0)
        pltpu.sync_copy(stage_v, deg_s)

    @pl.when(cid == 0)
    def _():
        pltpu.sync_copy(src_hbm.at[tid], idx_v)

    @pl.when(cid == 1)
    def _():
        pltpu.sync_copy(dst_hbm.at[tid], idx_v)

    plsc.subcore_barrier()

    def _chunk(c, carry):
        pltpu.sync_copy(ones_v, deg_s.at[idx_v.at[c]], add=True)
        return carry

    lax.fori_loop(0, NCH, _chunk, 0)
    plsc.subcore_barrier()

    @pl.when(tid == 0)
    def _():
        pltpu.sync_copy(deg_s, stage_v)

        @pl.when(cid == 0)
        def _():
            pltpu.sync_copy(stage_v, odeg_hbm)

        @pl.when(cid == 1)
        def _():
            pltpu.sync_copy(stage_v, ideg_hbm)


_deg = pl.kernel(
    _deg_body,
    out_type=[
        jax.ShapeDtypeStruct((N,), jnp.float32),
        jax.ShapeDtypeStruct((N,), jnp.float32),
    ],
    mesh=_MESH,
    scratch_types=[
        pltpu.VMEM((NCH, CH), jnp.int32),
        pltpu.VMEM((CH,), jnp.float32),
        pltpu.VMEM((N,), jnp.float32),
        pltpu.VMEM_SHARED((N,), jnp.float32),
    ],
)


def _agg_body(hw0_hbm, hw1_hbm, src_hbm, dst_hbm, zero_hbm,
              out0_hbm, out1_hbm,
              src_v, dst_v, buf_a, buf_b, agg_s, sem_a, sem_b):
    cid = lax.axis_index("c")
    tid = lax.axis_index("s")

    def _zero_acc(j, carry):
        c = tid + j * NT

        @pl.when(c < ZNC)
        def _():
            r = c * RCH
            pltpu.sync_copy(zero_hbm.at[pl.ds(r, RCH)],
                            agg_s.at[pl.ds(r, RCH)])

        return carry

    lax.fori_loop(0, RROUNDS, _zero_acc, 0)

    plsc.subcore_barrier()

    def _run(hw_hbm):
        def _seg(s, carry):
            pltpu.sync_copy(src_hbm.at[tid * SEG + s], src_v)
            pltpu.sync_copy(dst_hbm.at[tid * SEG + s], dst_v)

            def _pair(k, carry2):
                c0 = 2 * k
                c1 = 2 * k + 1
                cp0 = pltpu.async_copy(hw_hbm.at[src_v.at[c0]], buf_a,
                                       sem_a)
                cp1 = pltpu.async_copy(hw_hbm.at[src_v.at[c1]], buf_b,
                                       sem_b)
                cp0.wait()
                pltpu.sync_copy(buf_a, agg_s.at[dst_v.at[c0]], add=True)
                cp1.wait()
                pltpu.sync_copy(buf_b, agg_s.at[dst_v.at[c1]], add=True)
                return carry2

            lax.fori_loop(0, SEGC // 2, _pair, 0)
            cp = pltpu.async_copy(hw_hbm.at[src_v.at[SEGC - 1]], buf_a,
                                  sem_a)
            cp.wait()
            pltpu.sync_copy(buf_a, agg_s.at[dst_v.at[SEGC - 1]], add=True)
            return carry

        lax.fori_loop(0, SEG, _seg, 0)

    @pl.when(cid == 0)
    def _():
        _run(hw0_hbm)

    @pl.when(cid == 1)
    def _():
        _run(hw1_hbm)

    plsc.subcore_barrier()

    def _wb_loop(out_hbm):
        def _wb(j, carry):
            c = tid + j * NT

            @pl.when(c < ZNC)
            def _():
                r = c * RCH
                pltpu.sync_copy(agg_s.at[pl.ds(r, RCH)],
                                out_hbm.at[pl.ds(r, RCH)])

            return carry

        lax.fori_loop(0, RROUNDS, _wb, 0)

    @pl.when(cid == 0)
    def _():
        _wb_loop(out0_hbm)

    @pl.when(cid == 1)
    def _():
        _wb_loop(out1_hbm)


_agg = pl.kernel(
    _agg_body,
    out_type=[
        jax.ShapeDtypeStruct((N, HALF), jnp.float32),
        jax.ShapeDtypeStruct((N, HALF), jnp.float32),
    ],
    mesh=_MESH,
    scratch_types=[
        pltpu.VMEM((SEGC, CH), jnp.int32),
        pltpu.VMEM((SEGC, CH), jnp.int32),
        pltpu.VMEM((CH, HALF), jnp.float32),
        pltpu.VMEM((CH, HALF), jnp.float32),
        pltpu.VMEM_SHARED((N, HALF), jnp.float32),
        pltpu.SemaphoreType.DMA,
        pltpu.SemaphoreType.DMA,
    ],
)


_BR = 400
_NB = N // _BR


def _mm1_body(x_ref, deg_ref, w_ref, o0_ref, o1_ref):
    onorm = lax.rsqrt(jnp.maximum(deg_ref[...], 1.0))
    hw = jnp.dot(x_ref[...], w_ref[...],
                 preferred_element_type=jnp.float32) * onorm
    o0_ref[...] = hw[:, :HALF]
    o1_ref[...] = hw[:, HALF:]


def _mm1(x, deg_o, w1):
    return pl.pallas_call(
        _mm1_body,
        grid=(_NB,),
        in_specs=[
            pl.BlockSpec((_BR, D), lambda i: (i, 0)),
            pl.BlockSpec((_BR, 1), lambda i: (i, 0)),
            pl.BlockSpec((D, D), lambda i: (0, 0)),
        ],
        out_specs=[pl.BlockSpec((_BR, HALF), lambda i: (i, 0))] * 2,
        out_shape=[jax.ShapeDtypeStruct((N, HALF), jnp.float32)] * 2,
    )(x, deg_o, w1)


def _mm2_body(a0_ref, a1_ref, ideg_ref, odeg_ref, b_ref, w_ref,
              o0_ref, o1_ref):
    inorm = lax.rsqrt(jnp.maximum(ideg_ref[...], 1.0))
    onorm = lax.rsqrt(jnp.maximum(odeg_ref[...], 1.0))
    h0 = jnp.maximum(a0_ref[...] * inorm + b_ref[:, :HALF], 0.0)
    h1 = jnp.maximum(a1_ref[...] * inorm + b_ref[:, HALF:], 0.0)
    p = (jnp.dot(h0, w_ref[:HALF, :], preferred_element_type=jnp.float32)
         + jnp.dot(h1, w_ref[HALF:, :], preferred_element_type=jnp.float32))
    p = p * onorm
    o0_ref[...] = p[:, :HALF]
    o1_ref[...] = p[:, HALF:]


def _mm2(a0, a1, deg_i, deg_o, b1r, w2):
    return pl.pallas_call(
        _mm2_body,
        grid=(_NB,),
        in_specs=[
            pl.BlockSpec((_BR, HALF), lambda i: (i, 0)),
            pl.BlockSpec((_BR, HALF), lambda i: (i, 0)),
            pl.BlockSpec((_BR, 1), lambda i: (i, 0)),
            pl.BlockSpec((_BR, 1), lambda i: (i, 0)),
            pl.BlockSpec((1, D), lambda i: (0, 0)),
            pl.BlockSpec((D, D), lambda i: (0, 0)),
        ],
        out_specs=[pl.BlockSpec((_BR, HALF), lambda i: (i, 0))] * 2,
        out_shape=[jax.ShapeDtypeStruct((N, HALF), jnp.float32)] * 2,
    )(a0, a1, deg_i, deg_o, b1r, w2)


def _pool_body(a0_ref, a1_ref, ideg_ref, b_ref, o_ref):
    i = pl.program_id(0)
    inorm = lax.rsqrt(jnp.maximum(ideg_ref[...], 1.0))
    h0 = jnp.maximum(a0_ref[...] * inorm + b_ref[:, :HALF], 0.0)
    h1 = jnp.maximum(a1_ref[...] * inorm + b_ref[:, HALF:], 0.0)
    s = jnp.concatenate(
        [jnp.sum(h0, axis=0, keepdims=True),
         jnp.sum(h1, axis=0, keepdims=True)], axis=1)

    @pl.when(i == 0)
    def _():
        o_ref[...] = s

    @pl.when(i > 0)
    def _():
        o_ref[...] = o_ref[...] + s

    @pl.when(i == _NB - 1)
    def _():
        o_ref[...] = o_ref[...] * (1.0 / N)


def _pool(a0, a1, deg_i, b2r):
    return pl.pallas_call(
        _pool_body,
        grid=(_NB,),
        in_specs=[
            pl.BlockSpec((_BR, HALF), lambda i: (i, 0)),
            pl.BlockSpec((_BR, HALF), lambda i: (i, 0)),
            pl.BlockSpec((_BR, 1), lambda i: (i, 0)),
            pl.BlockSpec((1, D), lambda i: (0, 0)),
        ],
        out_specs=pl.BlockSpec((1, D), lambda i: (0, 0)),
        out_shape=jax.ShapeDtypeStruct((1, D), jnp.float32),
    )(a0, a1, deg_i, b2r)


def kernel(x, edge_index, W1, b1, W2, b2):
    assert x.shape == (N, D) and edge_index.shape == (2, E)
    src = edge_index[0].reshape(NT, NCH, CH)
    dst = edge_index[1].reshape(NT, NCH, CH)
    src_s = edge_index[0].reshape(NT * SEG, SEGC, CH)
    dst_s = edge_index[1].reshape(NT * SEG, SEGC, CH)

    odeg, ideg = _deg(src, dst)
    deg_o = odeg.reshape(N, 1)
    deg_i = ideg.reshape(N, 1)

    zeros = jnp.zeros((N, HALF), jnp.float32)
    hw0, hw1 = _mm1(x, deg_o, W1)
    a0, a1 = _agg(hw0, hw1, src_s, dst_s, zeros)
    g0, g1 = _mm2(a0, a1, deg_i, deg_o, b1.reshape(1, D), W2)
    c0, c1 = _agg(g0, g1, src_s, dst_s, zeros)
    return _pool(c0, c1, deg_i, b2.reshape(1, D))


# speedup vs baseline: 1.0134x; 1.0134x over previous
import jax
import jax.numpy as jnp
from jax import lax
from jax.experimental import pallas as pl
from jax.experimental.pallas import tpu as pltpu
from jax.experimental.pallas import tpu_sc as plsc

N = 10000
E = 160000
D = 256
HALF = 128

NT = 16
CH = 100
EPT = E // NT
NCH = EPT // CH
SEG = 5
SEGC = NCH // SEG
DCH = 80
DNCH = EPT // DCH
RCH = 80
ZNC = N // RCH
RROUNDS = -(-ZNC // NT)

_MESH = plsc.VectorSubcoreMesh(core_axis_name="c", subcore_axis_name="s")


def _deg_body(src_hbm, dst_hbm, odeg_hbm, ideg_hbm, idx_v, ones_v, stage_v,
              deg_s):
    cid = lax.axis_index("c")
    tid = lax.axis_index("s")

    for j in range(DCH // 16):
        ones_v[pl.ds(j * 16, 16)] = jnp.ones((16,), jnp.float32)

    @pl.when(tid == 0)
    def _():
        def _zero_stage(i, carry):
            stage_v[pl.ds(i * 16, 16)] = jnp.zeros((16,), jnp.float32)
            return carry

        lax.fori_loop(0, N // 16, _zero_stage, 0)
        pltpu.sync_copy(stage_v, deg_s)

    @pl.when(cid == 0)
    def _():
        pltpu.sync_copy(src_hbm.at[tid], idx_v)

    @pl.when(cid == 1)
    def _():
        pltpu.sync_copy(dst_hbm.at[tid], idx_v)

    plsc.subcore_barrier()

    def _chunk(c, carry):
        pltpu.sync_copy(ones_v, deg_s.at[idx_v.at[c]], add=True)
        return carry

    lax.fori_loop(0, DNCH, _chunk, 0)
    plsc.subcore_barrier()

    @pl.when(tid == 0)
    def _():
        pltpu.sync_copy(deg_s, stage_v)

        @pl.when(cid == 0)
        def _():
            pltpu.sync_copy(stage_v, odeg_hbm)

        @pl.when(cid == 1)
        def _():
            pltpu.sync_copy(stage_v, ideg_hbm)


_deg = pl.kernel(
    _deg_body,
    out_type=[
        jax.ShapeDtypeStruct((N,), jnp.float32),
        jax.ShapeDtypeStruct((N,), jnp.float32),
    ],
    mesh=_MESH,
    scratch_types=[
        pltpu.VMEM((DNCH, DCH), jnp.int32),
        pltpu.VMEM((DCH,), jnp.float32),
        pltpu.VMEM((N,), jnp.float32),
        pltpu.VMEM_SHARED((N,), jnp.float32),
    ],
)


def _agg_body(hw0_hbm, hw1_hbm, src_hbm, dst_hbm, out0_hbm, out1_hbm,
              src_v, dst_v, buf_a, buf_b, agg_s, sem_a, sem_b):
    cid = lax.axis_index("c")
    tid = lax.axis_index("s")

    def _zero_stage(i, carry):
        buf_a[i // 8, pl.ds((i % 8) * 16, 16)] = jnp.zeros((16,),
                                                           jnp.float32)
        return carry

    lax.fori_loop(0, CH * (HALF // 16), _zero_stage, 0)

    def _zero_acc(j, carry):
        c = tid + j * NT

        @pl.when(c < ZNC)
        def _():
            pltpu.sync_copy(buf_a.at[pl.ds(0, RCH)],
                            agg_s.at[pl.ds(c * RCH, RCH)])

        return carry

    lax.fori_loop(0, RROUNDS, _zero_acc, 0)

    plsc.subcore_barrier()

    def _run(hw_hbm):
        def _seg(s, carry):
            pltpu.sync_copy(src_hbm.at[tid * SEG + s], src_v)
            pltpu.sync_copy(dst_hbm.at[tid * SEG + s], dst_v)

            def _pair(k, carry2):
                c0 = 2 * k
                c1 = 2 * k + 1
                cp0 = pltpu.async_copy(hw_hbm.at[src_v.at[c0]], buf_a,
                                       sem_a)
                cp1 = pltpu.async_copy(hw_hbm.at[src_v.at[c1]], buf_b,
                                       sem_b)
                cp0.wait()
                pltpu.sync_copy(buf_a, agg_s.at[dst_v.at[c0]], add=True)
                cp1.wait()
                pltpu.sync_copy(buf_b, agg_s.at[dst_v.at[c1]], add=True)
                return carry2

            lax.fori_loop(0, SEGC // 2, _pair, 0)
            cp = pltpu.async_copy(hw_hbm.at[src_v.at[SEGC - 1]], buf_a,
                                  sem_a)
            cp.wait()
            pltpu.sync_copy(buf_a, agg_s.at[dst_v.at[SEGC - 1]], add=True)
            return carry

        lax.fori_loop(0, SEG, _seg, 0)

    @pl.when(cid == 0)
    def _():
        _run(hw0_hbm)

    @pl.when(cid == 1)
    def _():
        _run(hw1_hbm)

    plsc.subcore_barrier()

    def _wb_loop(out_hbm):
        def _wb(j, carry):
            c = tid + j * NT

            @pl.when(c < ZNC)
            def _():
                r = c * RCH
                pltpu.sync_copy(agg_s.at[pl.ds(r, RCH)],
                                out_hbm.at[pl.ds(r, RCH)])

            return carry

        lax.fori_loop(0, RROUNDS, _wb, 0)

    @pl.when(cid == 0)
    def _():
        _wb_loop(out0_hbm)

    @pl.when(cid == 1)
    def _():
        _wb_loop(out1_hbm)


_agg = pl.kernel(
    _agg_body,
    out_type=[
        jax.ShapeDtypeStruct((N, HALF), jnp.float32),
        jax.ShapeDtypeStruct((N, HALF), jnp.float32),
    ],
    mesh=_MESH,
    scratch_types=[
        pltpu.VMEM((SEGC, CH), jnp.int32),
        pltpu.VMEM((SEGC, CH), jnp.int32),
        pltpu.VMEM((CH, HALF), jnp.float32),
        pltpu.VMEM((CH, HALF), jnp.float32),
        pltpu.VMEM_SHARED((N, HALF), jnp.float32),
        pltpu.SemaphoreType.DMA,
        pltpu.SemaphoreType.DMA,
    ],
)


_BR = 400
_NB = N // _BR


def _mm1_body(x_ref, deg_ref, w_ref, o0_ref, o1_ref):
    onorm = lax.rsqrt(jnp.maximum(deg_ref[...], 1.0))
    hw = jnp.dot(x_ref[...], w_ref[...],
                 preferred_element_type=jnp.float32) * onorm
    o0_ref[...] = hw[:, :HALF]
    o1_ref[...] = hw[:, HALF:]


def _mm1(x, deg_o, w1):
    return pl.pallas_call(
        _mm1_body,
        grid=(_NB,),
        in_specs=[
            pl.BlockSpec((_BR, D), lambda i: (i, 0)),
            pl.BlockSpec((_BR, 1), lambda i: (i, 0)),
            pl.BlockSpec((D, D), lambda i: (0, 0)),
        ],
        out_specs=[pl.BlockSpec((_BR, HALF), lambda i: (i, 0))] * 2,
        out_shape=[jax.ShapeDtypeStruct((N, HALF), jnp.float32)] * 2,
    )(x, deg_o, w1)


def _mm2_body(a0_ref, a1_ref, ideg_ref, odeg_ref, b_ref, w_ref,
              o0_ref, o1_ref):
    inorm = lax.rsqrt(jnp.maximum(ideg_ref[...], 1.0))
    onorm = lax.rsqrt(jnp.maximum(odeg_ref[...], 1.0))
    h0 = jnp.maximum(a0_ref[...] * inorm + b_ref[:, :HALF], 0.0)
    h1 = jnp.maximum(a1_ref[...] * inorm + b_ref[:, HALF:], 0.0)
    p = (jnp.dot(h0, w_ref[:HALF, :], preferred_element_type=jnp.float32)
         + jnp.dot(h1, w_ref[HALF:, :], preferred_element_type=jnp.float32))
    p = p * onorm
    o0_ref[...] = p[:, :HALF]
    o1_ref[...] = p[:, HALF:]


def _mm2(a0, a1, deg_i, deg_o, b1r, w2):
    return pl.pallas_call(
        _mm2_body,
        grid=(_NB,),
        in_specs=[
            pl.BlockSpec((_BR, HALF), lambda i: (i, 0)),
            pl.BlockSpec((_BR, HALF), lambda i: (i, 0)),
            pl.BlockSpec((_BR, 1), lambda i: (i, 0)),
            pl.BlockSpec((_BR, 1), lambda i: (i, 0)),
            pl.BlockSpec((1, D), lambda i: (0, 0)),
            pl.BlockSpec((D, D), lambda i: (0, 0)),
        ],
        out_specs=[pl.BlockSpec((_BR, HALF), lambda i: (i, 0))] * 2,
        out_shape=[jax.ShapeDtypeStruct((N, HALF), jnp.float32)] * 2,
    )(a0, a1, deg_i, deg_o, b1r, w2)


def _pool_body(a0_ref, a1_ref, ideg_ref, b_ref, o_ref):
    i = pl.program_id(0)
    inorm = lax.rsqrt(jnp.maximum(ideg_ref[...], 1.0))
    h0 = jnp.maximum(a0_ref[...] * inorm + b_ref[:, :HALF], 0.0)
    h1 = jnp.maximum(a1_ref[...] * inorm + b_ref[:, HALF:], 0.0)
    s = jnp.concatenate(
        [jnp.sum(h0, axis=0, keepdims=True),
         jnp.sum(h1, axis=0, keepdims=True)], axis=1)

    @pl.when(i == 0)
    def _():
        o_ref[...] = s

    @pl.when(i > 0)
    def _():
        o_ref[...] = o_ref[...] + s

    @pl.when(i == _NB - 1)
    def _():
        o_ref[...] = o_ref[...] * (1.0 / N)


def _pool(a0, a1, deg_i, b2r):
    return pl.pallas_call(
        _pool_body,
        grid=(_NB,),
        in_specs=[
            pl.BlockSpec((_BR, HALF), lambda i: (i, 0)),
            pl.BlockSpec((_BR, HALF), lambda i: (i, 0)),
            pl.BlockSpec((_BR, 1), lambda i: (i, 0)),
            pl.BlockSpec((1, D), lambda i: (0, 0)),
        ],
        out_specs=pl.BlockSpec((1, D), lambda i: (0, 0)),
        out_shape=jax.ShapeDtypeStruct((1, D), jnp.float32),
    )(a0, a1, deg_i, b2r)


def kernel(x, edge_index, W1, b1, W2, b2):
    assert x.shape == (N, D) and edge_index.shape == (2, E)
    src = edge_index[0].reshape(NT, DNCH, DCH)
    dst = edge_index[1].reshape(NT, DNCH, DCH)
    src_s = edge_index[0].reshape(NT * SEG, SEGC, CH)
    dst_s = edge_index[1].reshape(NT * SEG, SEGC, CH)

    odeg, ideg = _deg(src, dst)
    deg_o = odeg.reshape(N, 1)
    deg_i = ideg.reshape(N, 1)

    hw0, hw1 = _mm1(x, deg_o, W1)
    a0, a1 = _agg(hw0, hw1, src_s, dst_s)
    g0, g1 = _mm2(a0, a1, deg_i, deg_o, b1.reshape(1, D), W2)
    c0, c1 = _agg(g0, g1, src_s, dst_s)
    return _pool(c0, c1, deg_i, b2.reshape(1, D))


# speedup vs baseline: 1.0544x; 1.0404x over previous
import jax
import jax.numpy as jnp
from jax import lax
from jax.experimental import pallas as pl
from jax.experimental.pallas import tpu as pltpu
from jax.experimental.pallas import tpu_sc as plsc

N = 10000
E = 160000
D = 256
HALF = 128

NT = 16
CH = 100
EPT = E // NT
NCH = EPT // CH
SEG = 5
SEGC = NCH // SEG
DCH = 80
DNCH = EPT // DCH
RCH = 80
ZNC = N // RCH
RROUNDS = -(-ZNC // NT)

_MESH = plsc.VectorSubcoreMesh(core_axis_name="c", subcore_axis_name="s")


def _deg_body(src_hbm, dst_hbm, odeg_hbm, ideg_hbm, idx_v, ones_v, stage_v,
              deg_s):
    cid = lax.axis_index("c")
    tid = lax.axis_index("s")

    for j in range(DCH // 16):
        ones_v[pl.ds(j * 16, 16)] = jnp.ones((16,), jnp.float32)

    @pl.when(tid == 0)
    def _():
        def _zero_stage(i, carry):
            stage_v[pl.ds(i * 16, 16)] = jnp.zeros((16,), jnp.float32)
            return carry

        lax.fori_loop(0, N // 16, _zero_stage, 0)
        pltpu.sync_copy(stage_v, deg_s)

    @pl.when(cid == 0)
    def _():
        pltpu.sync_copy(src_hbm.at[tid], idx_v)

    @pl.when(cid == 1)
    def _():
        pltpu.sync_copy(dst_hbm.at[tid], idx_v)

    plsc.subcore_barrier()

    def _chunk(c, carry):
        pltpu.sync_copy(ones_v, deg_s.at[idx_v.at[c]], add=True)
        return carry

    lax.fori_loop(0, DNCH, _chunk, 0)
    plsc.subcore_barrier()

    @pl.when(tid == 0)
    def _():
        pltpu.sync_copy(deg_s, stage_v)

        @pl.when(cid == 0)
        def _():
            pltpu.sync_copy(stage_v, odeg_hbm)

        @pl.when(cid == 1)
        def _():
            pltpu.sync_copy(stage_v, ideg_hbm)


_deg = pl.kernel(
    _deg_body,
    out_type=[
        jax.ShapeDtypeStruct((N,), jnp.float32),
        jax.ShapeDtypeStruct((N,), jnp.float32),
    ],
    mesh=_MESH,
    scratch_types=[
        pltpu.VMEM((DNCH, DCH), jnp.int32),
        pltpu.VMEM((DCH,), jnp.float32),
        pltpu.VMEM((N,), jnp.float32),
        pltpu.VMEM_SHARED((N,), jnp.float32),
    ],
)


def _agg_body(hw0_hbm, hw1_hbm, src_hbm, dst_hbm, out0_hbm, out1_hbm,
              src_v, dst_v, buf_a, buf_b, agg_s, sem_a, sem_b):
    cid = lax.axis_index("c")
    tid = lax.axis_index("s")

    def _zero_stage(i, carry):
        buf_a[i // 8, pl.ds((i % 8) * 16, 16)] = jnp.zeros((16,),
                                                           jnp.float32)
        return carry

    lax.fori_loop(0, CH * (HALF // 16), _zero_stage, 0)

    def _zero_acc(j, carry):
        c = tid + j * NT

        @pl.when(c < ZNC)
        def _():
            pltpu.sync_copy(buf_a.at[pl.ds(0, RCH)],
                            agg_s.at[pl.ds(c * RCH, RCH)])

        return carry

    lax.fori_loop(0, RROUNDS, _zero_acc, 0)

    plsc.subcore_barrier()

    def _run(hw_hbm):
        def _seg(s, carry):
            pltpu.sync_copy(src_hbm.at[tid * SEG + s], src_v)
            pltpu.sync_copy(dst_hbm.at[tid * SEG + s], dst_v)

            def _pair(k, carry2):
                c0 = 2 * k
                c1 = 2 * k + 1
                cp0 = pltpu.async_copy(hw_hbm.at[src_v.at[c0]], buf_a,
                                       sem_a)
                cp1 = pltpu.async_copy(hw_hbm.at[src_v.at[c1]], buf_b,
                                       sem_b)
                cp0.wait()
                pltpu.sync_copy(buf_a, agg_s.at[dst_v.at[c0]], add=True)
                cp1.wait()
                pltpu.sync_copy(buf_b, agg_s.at[dst_v.at[c1]], add=True)
                return carry2

            lax.fori_loop(0, SEGC // 2, _pair, 0)
            if SEGC % 2:
                cp = pltpu.async_copy(hw_hbm.at[src_v.at[SEGC - 1]],
                                      buf_a, sem_a)
                cp.wait()
                pltpu.sync_copy(buf_a, agg_s.at[dst_v.at[SEGC - 1]],
                                add=True)
            return carry

        lax.fori_loop(0, SEG, _seg, 0)

    @pl.when(cid == 0)
    def _():
        _run(hw0_hbm)

    @pl.when(cid == 1)
    def _():
        _run(hw1_hbm)

    plsc.subcore_barrier()

    def _wb_loop(out_hbm):
        def _wb(j, carry):
            c = tid + j * NT

            @pl.when(c < ZNC)
            def _():
                r = c * RCH
                pltpu.sync_copy(agg_s.at[pl.ds(r, RCH)],
                                out_hbm.at[pl.ds(r, RCH)])

            return carry

        lax.fori_loop(0, RROUNDS, _wb, 0)

    @pl.when(cid == 0)
    def _():
        _wb_loop(out0_hbm)

    @pl.when(cid == 1)
    def _():
        _wb_loop(out1_hbm)


_agg = pl.kernel(
    _agg_body,
    out_type=[
        jax.ShapeDtypeStruct((N, HALF), jnp.float32),
        jax.ShapeDtypeStruct((N, HALF), jnp.float32),
    ],
    mesh=_MESH,
    scratch_types=[
        pltpu.VMEM((SEGC, CH), jnp.int32),
        pltpu.VMEM((SEGC, CH), jnp.int32),
        pltpu.VMEM((CH, HALF), jnp.float32),
        pltpu.VMEM((CH, HALF), jnp.float32),
        pltpu.VMEM_SHARED((N, HALF), jnp.float32),
        pltpu.SemaphoreType.DMA,
        pltpu.SemaphoreType.DMA,
    ],
)


_BR = 400
_NB = N // _BR


def _mm1_body(x_ref, deg_ref, w_ref, o0_ref, o1_ref):
    onorm = lax.rsqrt(jnp.maximum(deg_ref[...], 1.0))
    hw = jnp.dot(x_ref[...], w_ref[...],
                 preferred_element_type=jnp.float32) * onorm
    o0_ref[...] = hw[:, :HALF]
    o1_ref[...] = hw[:, HALF:]


def _mm1(x, deg_o, w1):
    return pl.pallas_call(
        _mm1_body,
        grid=(_NB,),
        in_specs=[
            pl.BlockSpec((_BR, D), lambda i: (i, 0)),
            pl.BlockSpec((_BR, 1), lambda i: (i, 0)),
            pl.BlockSpec((D, D), lambda i: (0, 0)),
        ],
        out_specs=[pl.BlockSpec((_BR, HALF), lambda i: (i, 0))] * 2,
        out_shape=[jax.ShapeDtypeStruct((N, HALF), jnp.float32)] * 2,
    )(x, deg_o, w1)


def _mm2_body(a0_ref, a1_ref, ideg_ref, odeg_ref, b_ref, w_ref,
              o0_ref, o1_ref):
    inorm = lax.rsqrt(jnp.maximum(ideg_ref[...], 1.0))
    onorm = lax.rsqrt(jnp.maximum(odeg_ref[...], 1.0))
    h0 = jnp.maximum(a0_ref[...] * inorm + b_ref[:, :HALF], 0.0)
    h1 = jnp.maximum(a1_ref[...] * inorm + b_ref[:, HALF:], 0.0)
    p = (jnp.dot(h0, w_ref[:HALF, :], preferred_element_type=jnp.float32)
         + jnp.dot(h1, w_ref[HALF:, :], preferred_element_type=jnp.float32))
    p = p * onorm
    o0_ref[...] = p[:, :HALF]
    o1_ref[...] = p[:, HALF:]


def _mm2(a0, a1, deg_i, deg_o, b1r, w2):
    return pl.pallas_call(
        _mm2_body,
        grid=(_NB,),
        in_specs=[
            pl.BlockSpec((_BR, HALF), lambda i: (i, 0)),
            pl.BlockSpec((_BR, HALF), lambda i: (i, 0)),
            pl.BlockSpec((_BR, 1), lambda i: (i, 0)),
            pl.BlockSpec((_BR, 1), lambda i: (i, 0)),
            pl.BlockSpec((1, D), lambda i: (0, 0)),
            pl.BlockSpec((D, D), lambda i: (0, 0)),
        ],
        out_specs=[pl.BlockSpec((_BR, HALF), lambda i: (i, 0))] * 2,
        out_shape=[jax.ShapeDtypeStruct((N, HALF), jnp.float32)] * 2,
    )(a0, a1, deg_i, deg_o, b1r, w2)


def _pool_body(a0_ref, a1_ref, ideg_ref, b_ref, o_ref):
    i = pl.program_id(0)
    inorm = lax.rsqrt(jnp.maximum(ideg_ref[...], 1.0))
    h0 = jnp.maximum(a0_ref[...] * inorm + b_ref[:, :HALF], 0.0)
    h1 = jnp.maximum(a1_ref[...] * inorm + b_ref[:, HALF:], 0.0)
    s = jnp.concatenate(
        [jnp.sum(h0, axis=0, keepdims=True),
         jnp.sum(h1, axis=0, keepdims=True)], axis=1)

    @pl.when(i == 0)
    def _():
        o_ref[...] = s

    @pl.when(i > 0)
    def _():
        o_ref[...] = o_ref[...] + s

    @pl.when(i == _NB - 1)
    def _():
        o_ref[...] = o_ref[...] * (1.0 / N)


def _pool(a0, a1, deg_i, b2r):
    return pl.pallas_call(
        _pool_body,
        grid=(_NB,),
        in_specs=[
            pl.BlockSpec((_BR, HALF), lambda i: (i, 0)),
            pl.BlockSpec((_BR, HALF), lambda i: (i, 0)),
            pl.BlockSpec((_BR, 1), lambda i: (i, 0)),
            pl.BlockSpec((1, D), lambda i: (0, 0)),
        ],
        out_specs=pl.BlockSpec((1, D), lambda i: (0, 0)),
        out_shape=jax.ShapeDtypeStruct((1, D), jnp.float32),
    )(a0, a1, deg_i, b2r)


def kernel(x, edge_index, W1, b1, W2, b2):
    assert x.shape == (N, D) and edge_index.shape == (2, E)
    src = edge_index[0].reshape(NT, DNCH, DCH)
    dst = edge_index[1].reshape(NT, DNCH, DCH)
    src_s = edge_index[0].reshape(NT * SEG, SEGC, CH)
    dst_s = edge_index[1].reshape(NT * SEG, SEGC, CH)

    odeg, ideg = _deg(src, dst)
    deg_o = odeg.reshape(N, 1)
    deg_i = ideg.reshape(N, 1)

    hw0, hw1 = _mm1(x, deg_o, W1)
    a0, a1 = _agg(hw0, hw1, src_s, dst_s)
    g0, g1 = _mm2(a0, a1, deg_i, deg_o, b1.reshape(1, D), W2)
    c0, c1 = _agg(g0, g1, src_s, dst_s)
    return _pool(c0, c1, deg_i, b2.reshape(1, D))
